# baseline (speedup 1.0000x reference)
import jax
import jax.numpy as jnp
from jax import lax
from jax.experimental import pallas as pl
from jax.experimental.pallas import tpu as pltpu
from jax.experimental.pallas import tpu_sc as plsc

LANES = 16
NUM_CORES = 2
NUM_SUBCORES = 16
NUM_WORKERS = NUM_CORES * NUM_SUBCORES
LATENT = 32
PAIRS = LATENT // 2
BLK = 16384


def _packcopy_body(src_ref, dst_ref):
    x = src_ref[...].astype(jnp.bfloat16)
    lo = lax.convert_element_type(
        lax.bitcast_convert_type(x[:PAIRS, :], jnp.uint16), jnp.uint32)
    hi = lax.convert_element_type(
        lax.bitcast_convert_type(x[PAIRS:, :], jnp.uint16), jnp.uint32)
    packed = lax.bitcast_convert_type(
        lo | lax.shift_left(hi, jnp.uint32(16)), jnp.int32)
    packed = packed.reshape(PAIRS, BLK // 128, 128)
    packed = packed.transpose(1, 0, 2)
    dst_ref[...] = packed.reshape(BLK // 128 * PAIRS, 128)


def _packcopy(xt):
    n = xt.shape[1]
    nblocks = (n + BLK - 1) // BLK
    rows_per_blk = BLK // 128 * PAIRS
    return pl.pallas_call(
        _packcopy_body,
        grid=(nblocks,),
        in_specs=[pl.BlockSpec((LATENT, BLK), lambda j: (0, j))],
        out_specs=pl.BlockSpec((rows_per_blk, 128), lambda j: (j, 0)),
        out_shape=jax.ShapeDtypeStruct(
            (nblocks * rows_per_blk, 128), jnp.int32),
    )(xt)


def _svd_body(users_hbm, movies_hbm, Uf_hbm, Mf_hbm, ub_hbm, mb_hbm, out_hbm,
              uidx_v, midx_v, uflat_v, mflat_v, ucol_v, mcol_v,
              ubias_v, mbias_v, out_v, sem_b, sem_r):
    b_per_w = uidx_v.shape[0]
    wid = lax.axis_index("s") * NUM_CORES + lax.axis_index("c")
    base = wid * b_per_w

    pltpu.sync_copy(users_hbm.at[pl.ds(base, b_per_w)], uidx_v)
    pltpu.sync_copy(movies_hbm.at[pl.ds(base, b_per_w)], midx_v)

    bias_copies = [
        pltpu.async_copy(ub_hbm.at[uidx_v], ubias_v, sem_b),
        pltpu.async_copy(mb_hbm.at[midx_v], mbias_v, sem_b),
    ]

    def fill_idx(g, _):
        j0 = g * LANES
        sl = pl.ds(j0, LANES)
        uvec = uidx_v[sl]
        mvec = midx_v[sl]
        ub0 = lax.shift_left(lax.shift_right_logical(uvec, 7), 11) + (
            uvec & 127)
        mb0 = lax.shift_left(lax.shift_right_logical(mvec, 7), 11) + (
            mvec & 127)
        for dp in range(PAIRS):
            uflat_v[pl.ds(dp * b_per_w + j0, LANES)] = ub0 + (dp * 128)
            mflat_v[pl.ds(dp * b_per_w + j0, LANES)] = mb0 + (dp * 128)
        return 0

    lax.fori_loop(0, b_per_w // LANES, fill_idx, 0)

    QP = PAIRS // 4
    quarter_copies = []
    for q in range(4):
        qsl = pl.ds(q * QP * b_per_w, QP * b_per_w)
        quarter_copies.append((
            pltpu.async_copy(Uf_hbm.at[uflat_v.at[qsl]], ucol_v.at[qsl],
                             sem_r),
            pltpu.async_copy(Mf_hbm.at[mflat_v.at[qsl]], mcol_v.at[qsl],
                             sem_r),
        ))
    for cp in bias_copies:
        cp.wait()

    himask = jnp.full((LANES,), 0xFFFF0000, jnp.uint32).astype(jnp.int32)

    def init_out(g, _):
        sl = pl.ds(g * LANES, LANES)
        out_v[sl] = ubias_v[sl] + mbias_v[sl]
        return 0

    lax.fori_loop(0, b_per_w // LANES, init_out, 0, unroll=4)

    for q in range(4):
        for cp in quarter_copies[q]:
            cp.wait()

        def group(g, _):
            j0 = g * LANES
            sl = pl.ds(j0, LANES)
            acc = out_v[sl]
            for dp in range(q * QP, (q + 1) * QP):
                dsl = pl.ds(dp * b_per_w + j0, LANES)
                uw = ucol_v[dsl]
                mw = mcol_v[dsl]
                u0 = plsc.bitcast(lax.shift_left(uw, 16), jnp.float32)
                m0 = plsc.bitcast(lax.shift_left(mw, 16), jnp.float32)
                u1 = plsc.bitcast(uw & himask, jnp.float32)
                m1 = plsc.bitcast(mw & himask, jnp.float32)
                acc = acc + u0 * m0 + u1 * m1
            out_v[sl] = acc
            return 0

        lax.fori_loop(0, b_per_w // LANES, group, 0)

    pltpu.sync_copy(out_v, out_hbm.at[pl.ds(base, b_per_w)])


def kernel(users, movies, U, M, user_bias, movie_bias):
    B = users.shape[0]
    b_per_w = B // NUM_WORKERS
    users = users.astype(jnp.int32)
    movies = movies.astype(jnp.int32)
    Uf = _packcopy(U.T).reshape(-1)
    Mf = _packcopy(M.T).reshape(-1)
    mesh = plsc.VectorSubcoreMesh(core_axis_name="c", subcore_axis_name="s")
    k = pl.kernel(
        _svd_body,
        out_type=jax.ShapeDtypeStruct((B,), jnp.float32),
        mesh=mesh,
        compiler_params=pltpu.CompilerParams(needs_layout_passes=False),
        scratch_types=[
            pltpu.VMEM((b_per_w,), jnp.int32),
            pltpu.VMEM((b_per_w,), jnp.int32),
            pltpu.VMEM((PAIRS * b_per_w,), jnp.int32),
            pltpu.VMEM((PAIRS * b_per_w,), jnp.int32),
            pltpu.VMEM((PAIRS * b_per_w,), jnp.int32),
            pltpu.VMEM((PAIRS * b_per_w,), jnp.int32),
            pltpu.VMEM((b_per_w,), jnp.float32),
            pltpu.VMEM((b_per_w,), jnp.float32),
            pltpu.VMEM((b_per_w,), jnp.float32),
            pltpu.SemaphoreType.DMA,
            pltpu.SemaphoreType.DMA,
        ],
    )
    return k(users, movies, Uf, Mf, user_bias, movie_bias)


# speedup vs baseline: 1.1498x; 1.1498x over previous
import jax
import jax.numpy as jnp
from jax import lax
from jax.experimental import pallas as pl
from jax.experimental.pallas import tpu as pltpu
from jax.experimental.pallas import tpu_sc as plsc

LANES = 16
NUM_CORES = 2
NUM_SUBCORES = 16
NUM_WORKERS = NUM_CORES * NUM_SUBCORES
LATENT = 32
PAIRS = LATENT // 2
BLK = 32768


def _packcopy_body(src_ref, dst_ref):
    x = src_ref[...].astype(jnp.bfloat16)
    lo = lax.convert_element_type(
        lax.bitcast_convert_type(x[:PAIRS, :], jnp.uint16), jnp.uint32)
    hi = lax.convert_element_type(
        lax.bitcast_convert_type(x[PAIRS:, :], jnp.uint16), jnp.uint32)
    packed = lax.bitcast_convert_type(
        lo | lax.shift_left(hi, jnp.uint32(16)), jnp.int32)
    packed = packed.reshape(PAIRS, BLK // 128, 128)
    packed = packed.transpose(1, 0, 2)
    dst_ref[...] = packed.reshape(BLK // 128 * PAIRS, 128)


def _packcopy(xt):
    n = xt.shape[1]
    nblocks = (n + BLK - 1) // BLK
    rows_per_blk = BLK // 128 * PAIRS
    return pl.pallas_call(
        _packcopy_body,
        grid=(nblocks,),
        in_specs=[pl.BlockSpec((LATENT, BLK), lambda j: (0, j))],
        out_specs=pl.BlockSpec((rows_per_blk, 128), lambda j: (j, 0)),
        out_shape=jax.ShapeDtypeStruct(
            (nblocks * rows_per_blk, 128), jnp.int32),
    )(xt)


def _svd_body(users_hbm, movies_hbm, Uf_hbm, Mf_hbm, ub_hbm, mb_hbm, out_hbm,
              uidx_v, midx_v, uflat_v, mflat_v, ucol_v, mcol_v,
              ubias_v, mbias_v, out_v, sem_b, sem_r):
    b_per_w = uidx_v.shape[0]
    wid = lax.axis_index("s") * NUM_CORES + lax.axis_index("c")
    base = wid * b_per_w

    pltpu.sync_copy(users_hbm.at[pl.ds(base, b_per_w)], uidx_v)
    pltpu.sync_copy(movies_hbm.at[pl.ds(base, b_per_w)], midx_v)

    bias_copies = [
        pltpu.async_copy(ub_hbm.at[uidx_v], ubias_v, sem_b),
        pltpu.async_copy(mb_hbm.at[midx_v], mbias_v, sem_b),
    ]

    def fill_idx(g, _):
        j0 = g * LANES
        sl = pl.ds(j0, LANES)
        uvec = uidx_v[sl]
        mvec = midx_v[sl]
        ub0 = lax.shift_left(lax.shift_right_logical(uvec, 7), 11) + (
            uvec & 127)
        mb0 = lax.shift_left(lax.shift_right_logical(mvec, 7), 11) + (
            mvec & 127)
        for dp in range(PAIRS):
            uflat_v[pl.ds(dp * b_per_w + j0, LANES)] = ub0 + (dp * 128)
            mflat_v[pl.ds(dp * b_per_w + j0, LANES)] = mb0 + (dp * 128)
        return 0

    lax.fori_loop(0, b_per_w // LANES, fill_idx, 0)

    row_copies = [
        pltpu.async_copy(Uf_hbm.at[uflat_v], ucol_v, sem_r),
        pltpu.async_copy(Mf_hbm.at[mflat_v], mcol_v, sem_r),
    ]
    for cp in row_copies:
        cp.wait()
    for cp in bias_copies:
        cp.wait()

    himask = jnp.full((LANES,), 0xFFFF0000, jnp.uint32).astype(jnp.int32)

    def group(g, _):
        j0 = g * LANES
        sl = pl.ds(j0, LANES)
        acc = ubias_v[sl] + mbias_v[sl]
        for dp in range(PAIRS):
            dsl = pl.ds(dp * b_per_w + j0, LANES)
            uw = ucol_v[dsl]
            mw = mcol_v[dsl]
            u0 = plsc.bitcast(lax.shift_left(uw, 16), jnp.float32)
            m0 = plsc.bitcast(lax.shift_left(mw, 16), jnp.float32)
            u1 = plsc.bitcast(uw & himask, jnp.float32)
            m1 = plsc.bitcast(mw & himask, jnp.float32)
            acc = acc + u0 * m0 + u1 * m1
        out_v[sl] = acc
        return 0

    lax.fori_loop(0, b_per_w // LANES, group, 0)

    pltpu.sync_copy(out_v, out_hbm.at[pl.ds(base, b_per_w)])


def kernel(users, movies, U, M, user_bias, movie_bias):
    B = users.shape[0]
    b_per_w = B // NUM_WORKERS
    users = users.astype(jnp.int32)
    movies = movies.astype(jnp.int32)
    Uf = _packcopy(U.T).reshape(-1)
    Mf = _packcopy(M.T).reshape(-1)
    mesh = plsc.VectorSubcoreMesh(core_axis_name="c", subcore_axis_name="s")
    k = pl.kernel(
        _svd_body,
        out_type=jax.ShapeDtypeStruct((B,), jnp.float32),
        mesh=mesh,
        compiler_params=pltpu.CompilerParams(needs_layout_passes=False),
        scratch_types=[
            pltpu.VMEM((b_per_w,), jnp.int32),
            pltpu.VMEM((b_per_w,), jnp.int32),
            pltpu.VMEM((PAIRS * b_per_w,), jnp.int32),
            pltpu.VMEM((PAIRS * b_per_w,), jnp.int32),
            pltpu.VMEM((PAIRS * b_per_w,), jnp.int32),
            pltpu.VMEM((PAIRS * b_per_w,), jnp.int32),
            pltpu.VMEM((b_per_w,), jnp.float32),
            pltpu.VMEM((b_per_w,), jnp.float32),
            pltpu.VMEM((b_per_w,), jnp.float32),
            pltpu.SemaphoreType.DMA,
            pltpu.SemaphoreType.DMA,
        ],
    )
    return k(users, movies, Uf, Mf, user_bias, movie_bias)


# speedup vs baseline: 1.1609x; 1.0097x over previous
import jax
import jax.numpy as jnp
from jax import lax
from jax.experimental import pallas as pl
from jax.experimental.pallas import tpu as pltpu
from jax.experimental.pallas import tpu_sc as plsc

LANES = 16
NUM_CORES = 2
NUM_SUBCORES = 16
NUM_WORKERS = NUM_CORES * NUM_SUBCORES
LATENT = 32
PAIRS = LATENT // 2
BLK = 65536


def _packcopy_body(src_ref, dst_ref):
    x = src_ref[...].astype(jnp.bfloat16)
    lo = lax.convert_element_type(
        lax.bitcast_convert_type(x[:PAIRS, :], jnp.uint16), jnp.uint32)
    hi = lax.convert_element_type(
        lax.bitcast_convert_type(x[PAIRS:, :], jnp.uint16), jnp.uint32)
    packed = lax.bitcast_convert_type(
        lo | lax.shift_left(hi, jnp.uint32(16)), jnp.int32)
    packed = packed.reshape(PAIRS, BLK // 128, 128)
    packed = packed.transpose(1, 0, 2)
    dst_ref[...] = packed.reshape(BLK // 128 * PAIRS, 128)


def _packcopy(xt):
    n = xt.shape[1]
    nblocks = (n + BLK - 1) // BLK
    rows_per_blk = BLK // 128 * PAIRS
    return pl.pallas_call(
        _packcopy_body,
        grid=(nblocks,),
        in_specs=[pl.BlockSpec((LATENT, BLK), lambda j: (0, j))],
        out_specs=pl.BlockSpec((rows_per_blk, 128), lambda j: (j, 0)),
        out_shape=jax.ShapeDtypeStruct(
            (nblocks * rows_per_blk, 128), jnp.int32),
    )(xt)


def _svd_body(users_hbm, movies_hbm, Uf_hbm, Mf_hbm, ub_hbm, mb_hbm, out_hbm,
              uidx_v, midx_v, uflat_v, mflat_v, ucol_v, mcol_v,
              ubias_v, mbias_v, out_v, sem_b, sem_r):
    b_per_w = uidx_v.shape[0]
    wid = lax.axis_index("s") * NUM_CORES + lax.axis_index("c")
    base = wid * b_per_w

    pltpu.sync_copy(users_hbm.at[pl.ds(base, b_per_w)], uidx_v)
    pltpu.sync_copy(movies_hbm.at[pl.ds(base, b_per_w)], midx_v)

    bias_copies = [
        pltpu.async_copy(ub_hbm.at[uidx_v], ubias_v, sem_b),
        pltpu.async_copy(mb_hbm.at[midx_v], mbias_v, sem_b),
    ]

    def fill_idx(g, _):
        j0 = g * LANES
        sl = pl.ds(j0, LANES)
        uvec = uidx_v[sl]
        mvec = midx_v[sl]
        ub0 = lax.shift_left(lax.shift_right_logical(uvec, 7), 11) + (
            uvec & 127)
        mb0 = lax.shift_left(lax.shift_right_logical(mvec, 7), 11) + (
            mvec & 127)
        for dp in range(PAIRS):
            uflat_v[pl.ds(dp * b_per_w + j0, LANES)] = ub0 + (dp * 128)
            mflat_v[pl.ds(dp * b_per_w + j0, LANES)] = mb0 + (dp * 128)
        return 0

    lax.fori_loop(0, b_per_w // LANES, fill_idx, 0)

    row_copies = [
        pltpu.async_copy(Uf_hbm.at[uflat_v], ucol_v, sem_r),
        pltpu.async_copy(Mf_hbm.at[mflat_v], mcol_v, sem_r),
    ]
    for cp in row_copies:
        cp.wait()
    for cp in bias_copies:
        cp.wait()

    himask = jnp.full((LANES,), 0xFFFF0000, jnp.uint32).astype(jnp.int32)

    def group(g, _):
        j0 = g * LANES
        sl = pl.ds(j0, LANES)
        acc = ubias_v[sl] + mbias_v[sl]
        for dp in range(PAIRS):
            dsl = pl.ds(dp * b_per_w + j0, LANES)
            uw = ucol_v[dsl]
            mw = mcol_v[dsl]
            u0 = plsc.bitcast(lax.shift_left(uw, 16), jnp.float32)
            m0 = plsc.bitcast(lax.shift_left(mw, 16), jnp.float32)
            u1 = plsc.bitcast(uw & himask, jnp.float32)
            m1 = plsc.bitcast(mw & himask, jnp.float32)
            acc = acc + u0 * m0 + u1 * m1
        out_v[sl] = acc
        return 0

    lax.fori_loop(0, b_per_w // LANES, group, 0)

    pltpu.sync_copy(out_v, out_hbm.at[pl.ds(base, b_per_w)])


def kernel(users, movies, U, M, user_bias, movie_bias):
    B = users.shape[0]
    b_per_w = B // NUM_WORKERS
    users = users.astype(jnp.int32)
    movies = movies.astype(jnp.int32)
    Uf = _packcopy(U.T).reshape(-1)
    Mf = _packcopy(M.T).reshape(-1)
    mesh = plsc.VectorSubcoreMesh(core_axis_name="c", subcore_axis_name="s")
    k = pl.kernel(
        _svd_body,
        out_type=jax.ShapeDtypeStruct((B,), jnp.float32),
        mesh=mesh,
        compiler_params=pltpu.CompilerParams(needs_layout_passes=False),
        scratch_types=[
            pltpu.VMEM((b_per_w,), jnp.int32),
            pltpu.VMEM((b_per_w,), jnp.int32),
            pltpu.VMEM((PAIRS * b_per_w,), jnp.int32),
            pltpu.VMEM((PAIRS * b_per_w,), jnp.int32),
            pltpu.VMEM((PAIRS * b_per_w,), jnp.int32),
            pltpu.VMEM((PAIRS * b_per_w,), jnp.int32),
            pltpu.VMEM((b_per_w,), jnp.float32),
            pltpu.VMEM((b_per_w,), jnp.float32),
            pltpu.VMEM((b_per_w,), jnp.float32),
            pltpu.SemaphoreType.DMA,
            pltpu.SemaphoreType.DMA,
        ],
    )
    return k(users, movies, Uf, Mf, user_bias, movie_bias)
